# baseline (speedup 1.0000x reference)
import numpy as np
import jax
import jax.numpy as jnp
from jax import lax
from jax.experimental import pallas as pl
from jax.experimental.pallas import tpu as pltpu
from jax.experimental.pallas import tpu_sc as plsc

B, C, H, W = 4, 3, 224, 224
K = 16
RFS = 4
STRIDE = 4
OUT_H = (H - RFS) // STRIDE + 1
OUT_W = (W - RFS) // STRIDE + 1
N0 = 4

BANDS = 7
ROWS_PER_BAND = OUT_H // BANDS
NW_USED = B * BANDS
SLAB_ROWS = ROWS_PER_BAND * STRIDE


def _static_taps():
    rng = np.random.default_rng(0)
    dh = rng.integers(0, RFS, size=(2, N0, K))
    dw = rng.integers(0, RFS, size=(2, N0, K))
    dc = rng.integers(0, C, size=(2, N0, K))
    tap_rc = (dc * SLAB_ROWS + dh).reshape(2 * N0, K)
    tap_w = dw.reshape(2 * N0, K)
    return jnp.asarray(np.concatenate([tap_rc, tap_w], axis=0)
                       .astype(np.int32))


TAPS = _static_taps()

_OP_AFFINE = (
    (0, 0, 0, 0), (0, 0, 0, 1), (0, 1, 0, -1), (0, 1, 0, 0),
    (0, 0, 1, -1), (0, 0, 1, 0), (0, 1, 1, -2), (0, 1, 1, -1),
    (1, -1, -1, 1), (1, -1, -1, 2), (1, 0, -1, 0), (1, 0, -1, 1),
    (1, -1, 0, 0), (1, -1, 0, 1), (1, 0, 0, -1), (1, 0, 0, 0),
)


def _sc_kernel(x_hbm, pk_hbm, out_hbm, slab, pk_v, outbuf, dma_sem):
    wid = lax.axis_index("s") * 2 + lax.axis_index("c")

    @pl.when(wid < NW_USED)
    def _body():
        b = wid // BANDS
        band = wid % BANDS
        oh0 = band * ROWS_PER_BAND

        slab_cps = [
            pltpu.async_copy(
                x_hbm.at[b, c, pl.ds(oh0 * STRIDE, SLAB_ROWS), :],
                slab.at[pl.ds(c * SLAB_ROWS, SLAB_ROWS), :],
                dma_sem)
            for c in range(C)
        ]
        pltpu.sync_copy(pk_hbm, pk_v)

        coefs = []
        for node in range(7):
            v = [pk_v[j, pl.ds(node * K, K)] for j in range(16)]
            m = v[0]
            for j in range(1, 16):
                m = jnp.maximum(m, v[j])
            e = [jnp.exp(v[j] - m) for j in range(16)]
            s = e[0]
            for j in range(1, 16):
                s = s + e[j]
            inv = 1.0 / s
            cf = []
            for c_idx in range(4):
                acc = None
                for j in range(16):
                    w_j = float(_OP_AFFINE[j][c_idx])
                    if w_j == 0.0:
                        continue
                    term = e[j] if w_j == 1.0 else w_j * e[j]
                    acc = term if acc is None else acc + term
                cf.append(acc * inv)
            coefs.append(cf)

        tap = [pk_v[t, pl.ds(7 * K, K)].astype(jnp.int32)
               for t in range(4 * N0)]
        rc, cw = tap[:2 * N0], tap[2 * N0:]
        lane = lax.iota(jnp.int32, 16)

        for cp in slab_cps:
            cp.wait()

        def binop(a, bb, cf):
            c0, c1, c2, c3 = cf
            return (c0 + c2 * bb) + a * (c1 + c3 * bb)

        UNROLL = 2

        def row_body(loh, _):
            rowoff = loh * STRIDE
            rrc = [rc[t] + rowoff for t in range(2 * N0)]
            sloh = jnp.full((16,), loh, jnp.int32)

            def col_body(cb, _):
                ow0 = cb * UNROLL
                sow0 = jnp.full((16,), ow0, jnp.int32)
                for u in range(UNROLL):
                    coloff = (ow0 + u) * STRIDE
                    cur = []
                    for n in range(N0):
                        a = plsc.load_gather(slab, [rrc[n], cw[n] + coloff])
                        bb = plsc.load_gather(
                            slab, [rrc[N0 + n], cw[N0 + n] + coloff])
                        cur.append(binop(a, bb, coefs[n]))
                    lvl1 = [binop(cur[0], cur[1], coefs[4]),
                            binop(cur[2], cur[3], coefs[5])]
                    val = binop(lvl1[0], lvl1[1], coefs[6])
                    plsc.store_scatter(outbuf, [lane, sloh, sow0 + u], val)
                return _

            return lax.fori_loop(0, OUT_W // UNROLL, col_body, _)

        lax.fori_loop(0, ROWS_PER_BAND, row_body, 0)

        pltpu.sync_copy(outbuf, out_hbm.at[b, :, pl.ds(oh0, ROWS_PER_BAND), :])


def kernel(x, w0, w1, w2):
    wt = jnp.transpose(jnp.concatenate([w0, w1, w2], axis=0),
                       (2, 0, 1)).reshape(16, 7 * K)
    pk = jnp.concatenate([wt, TAPS.astype(jnp.float32)], axis=1)

    mesh = plsc.VectorSubcoreMesh(core_axis_name="c", subcore_axis_name="s")
    run = pl.kernel(
        _sc_kernel,
        out_type=jax.ShapeDtypeStruct((B, K, OUT_H, OUT_W), jnp.float32),
        mesh=mesh,
        scratch_types=[
            pltpu.VMEM((C * SLAB_ROWS, W), jnp.float32),
            pltpu.VMEM((16, 8 * K), jnp.float32),
            pltpu.VMEM((K, ROWS_PER_BAND, OUT_W), jnp.float32),
            pltpu.SemaphoreType.DMA,
        ],
        compiler_params=pltpu.CompilerParams(
            use_tc_tiling_on_sc=False, needs_layout_passes=False),
    )
    return run(x, pk)


# speedup vs baseline: 1.1760x; 1.1760x over previous
import numpy as np
import jax
import jax.numpy as jnp
from jax import lax
from jax.experimental import pallas as pl
from jax.experimental.pallas import tpu as pltpu
from jax.experimental.pallas import tpu_sc as plsc

B, C, H, W = 4, 3, 224, 224
K = 16
RFS = 4
STRIDE = 4
OUT_H = (H - RFS) // STRIDE + 1
OUT_W = (W - RFS) // STRIDE + 1
N0 = 4

BANDS = 7
ROWS_PER_BAND = OUT_H // BANDS
NW_USED = B * BANDS
SLAB_ROWS = ROWS_PER_BAND * STRIDE


def _static_taps():
    rng = np.random.default_rng(0)
    dh = rng.integers(0, RFS, size=(2, N0, K))
    dw = rng.integers(0, RFS, size=(2, N0, K))
    dc = rng.integers(0, C, size=(2, N0, K))
    tap_rc = (dc * SLAB_ROWS + dh).reshape(2 * N0, K)
    tap_w = dw.reshape(2 * N0, K)
    return jnp.asarray(np.concatenate([tap_rc, tap_w], axis=0)
                       .astype(np.int32))


TAPS = _static_taps()

_OP_AFFINE = (
    (0, 0, 0, 0), (0, 0, 0, 1), (0, 1, 0, -1), (0, 1, 0, 0),
    (0, 0, 1, -1), (0, 0, 1, 0), (0, 1, 1, -2), (0, 1, 1, -1),
    (1, -1, -1, 1), (1, -1, -1, 2), (1, 0, -1, 0), (1, 0, -1, 1),
    (1, -1, 0, 0), (1, -1, 0, 1), (1, 0, 0, -1), (1, 0, 0, 0),
)


def _sc_kernel(x_hbm, pk_hbm, out_hbm, slab, pk_v, outbuf, dma_sem):
    wid = lax.axis_index("s") * 2 + lax.axis_index("c")

    @pl.when(wid < NW_USED)
    def _body():
        b = wid // BANDS
        band = wid % BANDS
        oh0 = band * ROWS_PER_BAND

        slab_cps = [
            pltpu.async_copy(
                x_hbm.at[b, c, pl.ds(oh0 * STRIDE, SLAB_ROWS), :],
                slab.at[pl.ds(c * SLAB_ROWS, SLAB_ROWS), :],
                dma_sem)
            for c in range(C)
        ]
        pltpu.sync_copy(pk_hbm, pk_v)

        coefs = []
        for node in range(7):
            v = [pk_v[j, pl.ds(node * K, K)] for j in range(16)]
            m = v[0]
            for j in range(1, 16):
                m = jnp.maximum(m, v[j])
            e = [jnp.exp(v[j] - m) for j in range(16)]
            s = e[0]
            for j in range(1, 16):
                s = s + e[j]
            inv = 1.0 / s
            cf = []
            for c_idx in range(4):
                acc = None
                for j in range(16):
                    w_j = float(_OP_AFFINE[j][c_idx])
                    if w_j == 0.0:
                        continue
                    term = e[j] if w_j == 1.0 else w_j * e[j]
                    acc = term if acc is None else acc + term
                cf.append(acc * inv)
            coefs.append(cf)

        tap = [pk_v[t, pl.ds(7 * K, K)].astype(jnp.int32)
               for t in range(4 * N0)]
        rc, cw = tap[:2 * N0], tap[2 * N0:]
        lane = lax.iota(jnp.int32, 16)

        for cp in slab_cps:
            cp.wait()

        def binop(a, bb, cf):
            c0, c1, c2, c3 = cf
            return (c0 + c2 * bb) + a * (c1 + c3 * bb)

        UNROLL = 14

        def row_body(loh, _):
            rowoff = loh * STRIDE
            rrc = [rc[t] + rowoff for t in range(2 * N0)]
            sloh = jnp.full((16,), loh, jnp.int32)

            def col_body(cb, _):
                ow0 = cb * UNROLL
                sow0 = jnp.full((16,), ow0, jnp.int32)
                for u in range(UNROLL):
                    coloff = (ow0 + u) * STRIDE
                    cur = []
                    for n in range(N0):
                        a = plsc.load_gather(slab, [rrc[n], cw[n] + coloff])
                        bb = plsc.load_gather(
                            slab, [rrc[N0 + n], cw[N0 + n] + coloff])
                        cur.append(binop(a, bb, coefs[n]))
                    lvl1 = [binop(cur[0], cur[1], coefs[4]),
                            binop(cur[2], cur[3], coefs[5])]
                    val = binop(lvl1[0], lvl1[1], coefs[6])
                    plsc.store_scatter(outbuf, [lane, sloh, sow0 + u], val)
                return _

            return lax.fori_loop(0, OUT_W // UNROLL, col_body, _)

        lax.fori_loop(0, ROWS_PER_BAND, row_body, 0)

        pltpu.sync_copy(outbuf, out_hbm.at[b, :, pl.ds(oh0, ROWS_PER_BAND), :])


def kernel(x, w0, w1, w2):
    wt = jnp.transpose(jnp.concatenate([w0, w1, w2], axis=0),
                       (2, 0, 1)).reshape(16, 7 * K)
    pk = jnp.concatenate([wt, TAPS.astype(jnp.float32)], axis=1)

    mesh = plsc.VectorSubcoreMesh(core_axis_name="c", subcore_axis_name="s")
    run = pl.kernel(
        _sc_kernel,
        out_type=jax.ShapeDtypeStruct((B, K, OUT_H, OUT_W), jnp.float32),
        mesh=mesh,
        scratch_types=[
            pltpu.VMEM((C * SLAB_ROWS, W), jnp.float32),
            pltpu.VMEM((16, 8 * K), jnp.float32),
            pltpu.VMEM((K, ROWS_PER_BAND, OUT_W), jnp.float32),
            pltpu.SemaphoreType.DMA,
        ],
        compiler_params=pltpu.CompilerParams(
            use_tc_tiling_on_sc=False, needs_layout_passes=False),
    )
    return run(x, pk)


# speedup vs baseline: 1.2367x; 1.0516x over previous
import numpy as np
import jax
import jax.numpy as jnp
from jax import lax
from jax.experimental import pallas as pl
from jax.experimental.pallas import tpu as pltpu
from jax.experimental.pallas import tpu_sc as plsc

B, C, H, W = 4, 3, 224, 224
K = 16
RFS = 4
STRIDE = 4
OUT_H = (H - RFS) // STRIDE + 1
OUT_W = (W - RFS) // STRIDE + 1
N0 = 4

BANDS = 7
ROWS_PER_BAND = OUT_H // BANDS
NW_USED = B * BANDS
SLAB_ROWS = ROWS_PER_BAND * STRIDE


def _static_taps():
    rng = np.random.default_rng(0)
    dh = rng.integers(0, RFS, size=(2, N0, K))
    dw = rng.integers(0, RFS, size=(2, N0, K))
    dc = rng.integers(0, C, size=(2, N0, K))
    tap_rc = (dc * SLAB_ROWS + dh).reshape(2 * N0, K)
    tap_w = dw.reshape(2 * N0, K)
    return jnp.asarray(np.concatenate([tap_rc, tap_w], axis=0)
                       .astype(np.int32))


TAPS = _static_taps()

_OP_AFFINE = (
    (0, 0, 0, 0), (0, 0, 0, 1), (0, 1, 0, -1), (0, 1, 0, 0),
    (0, 0, 1, -1), (0, 0, 1, 0), (0, 1, 1, -2), (0, 1, 1, -1),
    (1, -1, -1, 1), (1, -1, -1, 2), (1, 0, -1, 0), (1, 0, -1, 1),
    (1, -1, 0, 0), (1, -1, 0, 1), (1, 0, 0, -1), (1, 0, 0, 0),
)


def _sc_kernel(x_hbm, pk_hbm, out_hbm, slab, pk_v, outbuf, dma_sem):
    wid = lax.axis_index("s") * 2 + lax.axis_index("c")

    @pl.when(wid < NW_USED)
    def _body():
        b = wid // BANDS
        band = wid % BANDS
        oh0 = band * ROWS_PER_BAND

        slab_cps = [
            pltpu.async_copy(
                x_hbm.at[b, c, pl.ds(oh0 * STRIDE, SLAB_ROWS), :],
                slab.at[pl.ds(c * SLAB_ROWS, SLAB_ROWS), :],
                dma_sem)
            for c in range(C)
        ]
        pltpu.sync_copy(pk_hbm, pk_v)

        coefs = []
        for node in range(7):
            v = [pk_v[j, pl.ds(node * K, K)] for j in range(16)]
            m = v[0]
            for j in range(1, 16):
                m = jnp.maximum(m, v[j])
            e = [jnp.exp(v[j] - m) for j in range(16)]
            s = e[0]
            for j in range(1, 16):
                s = s + e[j]
            inv = 1.0 / s
            cf = []
            for c_idx in range(4):
                acc = None
                for j in range(16):
                    w_j = float(_OP_AFFINE[j][c_idx])
                    if w_j == 0.0:
                        continue
                    term = e[j] if w_j == 1.0 else w_j * e[j]
                    acc = term if acc is None else acc + term
                cf.append(acc * inv)
            coefs.append(cf)

        tap = [pk_v[t, pl.ds(7 * K, K)].astype(jnp.int32)
               for t in range(4 * N0)]
        rc, cw = tap[:2 * N0], tap[2 * N0:]
        lane = lax.iota(jnp.int32, 16)

        for cp in slab_cps:
            cp.wait()

        def binop(a, bb, cf):
            c0, c1, c2, c3 = cf
            return (c0 + c2 * bb) + a * (c1 + c3 * bb)

        UNROLL = 8

        def row_body(loh, _):
            rowoff = loh * STRIDE
            rrc = [rc[t] + rowoff for t in range(2 * N0)]
            sloh = jnp.full((16,), loh, jnp.int32)

            def col_body(cb, _):
                ow0 = cb * UNROLL
                sow0 = jnp.full((16,), ow0, jnp.int32)
                for u in range(UNROLL):
                    coloff = (ow0 + u) * STRIDE
                    cur = []
                    for n in range(N0):
                        a = plsc.load_gather(slab, [rrc[n], cw[n] + coloff])
                        bb = plsc.load_gather(
                            slab, [rrc[N0 + n], cw[N0 + n] + coloff])
                        cur.append(binop(a, bb, coefs[n]))
                    lvl1 = [binop(cur[0], cur[1], coefs[4]),
                            binop(cur[2], cur[3], coefs[5])]
                    val = binop(lvl1[0], lvl1[1], coefs[6])
                    plsc.store_scatter(outbuf, [lane, sloh, sow0 + u], val)
                return _

            return lax.fori_loop(0, OUT_W // UNROLL, col_body, _)

        lax.fori_loop(0, ROWS_PER_BAND, row_body, 0)

        pltpu.sync_copy(outbuf, out_hbm.at[b, :, pl.ds(oh0, ROWS_PER_BAND), :])


def kernel(x, w0, w1, w2):
    wt = jnp.transpose(jnp.concatenate([w0, w1, w2], axis=0),
                       (2, 0, 1)).reshape(16, 7 * K)
    pk = jnp.concatenate([wt, TAPS.astype(jnp.float32)], axis=1)

    mesh = plsc.VectorSubcoreMesh(core_axis_name="c", subcore_axis_name="s")
    run = pl.kernel(
        _sc_kernel,
        out_type=jax.ShapeDtypeStruct((B, K, OUT_H, OUT_W), jnp.float32),
        mesh=mesh,
        scratch_types=[
            pltpu.VMEM((C * SLAB_ROWS, W), jnp.float32),
            pltpu.VMEM((16, 8 * K), jnp.float32),
            pltpu.VMEM((K, ROWS_PER_BAND, OUT_W), jnp.float32),
            pltpu.SemaphoreType.DMA,
        ],
        compiler_params=pltpu.CompilerParams(
            use_tc_tiling_on_sc=False, needs_layout_passes=False),
    )
    return run(x, pk)
